# baseline (speedup 1.0000x reference)
import functools

import jax
import jax.numpy as jnp
from jax.experimental import pallas as pl
from jax.experimental.pallas import tpu as pltpu


def _unpack_bits(pk):
    pieces = [((pk >> k) & 1).astype(jnp.float32) for k in range(32)]
    return jnp.concatenate(pieces, axis=1)


def _pack_bits(B, nwords):
    lo = jnp.zeros((B.shape[0], nwords), jnp.float32)
    hi = jnp.zeros((B.shape[0], nwords), jnp.float32)
    for k in range(16):
        lo = lo + B[:, k * nwords:(k + 1) * nwords] * float(1 << k)
    for k in range(16, 32):
        hi = hi + B[:, k * nwords:(k + 1) * nwords] * float(1 << (k - 16))
    return lo.astype(jnp.int32) | (hi.astype(jnp.int32) << 16)


def _pass1_body(x0_ref, b_ref, win_ref, bin_ref, x1_ref, pk_ref, m1_ref):
    i = pl.program_id(0)
    B = b_ref[...]
    pk_ref[...] = _pack_bits(B, pk_ref.shape[1])
    x1 = jnp.dot(x0_ref[...], win_ref[...], preferred_element_type=jnp.float32) + bin_ref[...]
    x1_ref[...] = x1

    @pl.when(i == 0)
    def _():
        m1_ref[...] = jnp.zeros_like(m1_ref)

    m1_ref[...] += jax.lax.dot_general(
        B, x1, (((0,), (0,)), ((), ())), preferred_element_type=jnp.float32)


def _pass2_body(n_total, nb, pk_ref, x1_ref, m1_ref,
                w1a_ref, b1a_ref, w2a_ref, b2a_ref,
                w1b_ref, b1b_ref, w2b_ref, b2b_ref,
                wout_ref, bout_ref, out_ref,
                x2_buf, macc, pool_ref):
    p = pl.program_id(0)
    i = pl.program_id(1)
    Bf = _unpack_bits(pk_ref[...])

    @pl.when(p == 0)
    def _layer1():
        y = x1_ref[...] + jnp.dot(Bf, m1_ref[...], preferred_element_type=jnp.float32)
        h = jnp.maximum(jnp.dot(y, w1a_ref[...], preferred_element_type=jnp.float32) + b1a_ref[...], 0.0)
        x2 = jnp.dot(h, w2a_ref[...], preferred_element_type=jnp.float32) + b2a_ref[...]
        x2_buf[pl.ds(i * nb, nb), :] = x2

        @pl.when(i == 0)
        def _():
            macc[...] = jnp.zeros_like(macc)

        macc[...] += jax.lax.dot_general(
            Bf, x2, (((0,), (0,)), ((), ())), preferred_element_type=jnp.float32)

    @pl.when(p == 1)
    def _layer2():
        y = x2_buf[pl.ds(i * nb, nb), :] + jnp.dot(Bf, macc[...], preferred_element_type=jnp.float32)
        h = jnp.maximum(jnp.dot(y, w1b_ref[...], preferred_element_type=jnp.float32) + b1b_ref[...], 0.0)
        x3 = jnp.dot(h, w2b_ref[...], preferred_element_type=jnp.float32) + b2b_ref[...]

        @pl.when(i == 0)
        def _():
            pool_ref[...] = jnp.zeros_like(pool_ref)

        pool_ref[...] += jnp.sum(x3, axis=0, keepdims=True)

        @pl.when(i == pl.num_programs(1) - 1)
        def _():
            pooled = pool_ref[...] * (1.0 / n_total)
            z = jnp.dot(pooled, wout_ref[...], preferred_element_type=jnp.float32) + bout_ref[...]
            out_ref[...] = jax.nn.sigmoid(z)


def kernel(x_0, incidence_1, W_in, b_in, W1_0, b1_0, W2_0, b2_0,
           W1_1, b1_1, W2_1, b2_1, W_out, b_out):
    N, D = x_0.shape
    E = incidence_1.shape[1]
    EPK = E // 32
    D2 = W1_0.shape[1]
    NB = 1000 if N % 1000 == 0 else 400
    grid = (N // NB,)

    b_in2 = b_in.reshape(1, -1)
    b1_02 = b1_0.reshape(1, -1)
    b2_02 = b2_0.reshape(1, -1)
    b1_12 = b1_1.reshape(1, -1)
    b2_12 = b2_1.reshape(1, -1)
    b_out2 = b_out.reshape(1, -1)

    x1, pk, m1 = pl.pallas_call(
        _pass1_body,
        grid=grid,
        in_specs=[
            pl.BlockSpec((NB, D), lambda i: (i, 0)),
            pl.BlockSpec((NB, E), lambda i: (i, 0)),
            pl.BlockSpec((D, D), lambda i: (0, 0)),
            pl.BlockSpec((1, D), lambda i: (0, 0)),
        ],
        out_specs=[
            pl.BlockSpec((NB, D), lambda i: (i, 0)),
            pl.BlockSpec((NB, EPK), lambda i: (i, 0)),
            pl.BlockSpec((E, D), lambda i: (0, 0)),
        ],
        out_shape=[
            jax.ShapeDtypeStruct((N, D), jnp.float32),
            jax.ShapeDtypeStruct((N, EPK), jnp.int32),
            jax.ShapeDtypeStruct((E, D), jnp.float32),
        ],
        compiler_params=pltpu.CompilerParams(dimension_semantics=("arbitrary",)),
    )(x_0, incidence_1, W_in, b_in2)

    out = pl.pallas_call(
        functools.partial(_pass2_body, float(N), NB),
        grid=(2, N // NB),
        in_specs=[
            pl.BlockSpec((NB, EPK), lambda p, i: (i, 0)),
            pl.BlockSpec((NB, D), lambda p, i: (i, 0)),
            pl.BlockSpec((E, D), lambda p, i: (0, 0)),
            pl.BlockSpec((D, D2), lambda p, i: (0, 0)),
            pl.BlockSpec((1, D2), lambda p, i: (0, 0)),
            pl.BlockSpec((D2, D), lambda p, i: (0, 0)),
            pl.BlockSpec((1, D), lambda p, i: (0, 0)),
            pl.BlockSpec((D, D2), lambda p, i: (0, 0)),
            pl.BlockSpec((1, D2), lambda p, i: (0, 0)),
            pl.BlockSpec((D2, D), lambda p, i: (0, 0)),
            pl.BlockSpec((1, D), lambda p, i: (0, 0)),
            pl.BlockSpec((D, W_out.shape[1]), lambda p, i: (0, 0)),
            pl.BlockSpec((1, W_out.shape[1]), lambda p, i: (0, 0)),
        ],
        out_specs=pl.BlockSpec((1, W_out.shape[1]), lambda p, i: (0, 0)),
        out_shape=jax.ShapeDtypeStruct((1, W_out.shape[1]), jnp.float32),
        scratch_shapes=[
            pltpu.VMEM((N, D), jnp.float32),
            pltpu.VMEM((E, D), jnp.float32),
            pltpu.VMEM((1, D), jnp.float32),
        ],
        compiler_params=pltpu.CompilerParams(
            dimension_semantics=("arbitrary", "arbitrary")),
    )(pk, x1, m1, W1_0, b1_02, W2_0, b2_02, W1_1, b1_12, W2_1, b2_12,
      W_out, b_out2)

    return out.reshape(-1)


# speedup vs baseline: 1.5823x; 1.5823x over previous
import functools

import jax
import jax.numpy as jnp
from jax.experimental import pallas as pl
from jax.experimental.pallas import tpu as pltpu


def _unpack_bits(pk):
    pieces = [((pk >> k) & 1).astype(jnp.bfloat16) for k in range(32)]
    return jnp.concatenate(pieces, axis=1)


def _pack_bits(B, nwords):
    lo = jnp.zeros((B.shape[0], nwords), jnp.float32)
    hi = jnp.zeros((B.shape[0], nwords), jnp.float32)
    for k in range(16):
        lo = lo + B[:, k * nwords:(k + 1) * nwords] * float(1 << k)
    for k in range(16, 32):
        hi = hi + B[:, k * nwords:(k + 1) * nwords] * float(1 << (k - 16))
    return lo.astype(jnp.int32) | (hi.astype(jnp.int32) << 16)


def _pass1_body(x0_ref, b_ref, win_ref, bin_ref, x1_ref, pk_ref, m1_ref, acc_ref):
    i = pl.program_id(0)
    B = b_ref[...]
    pk_ref[...] = _pack_bits(B, pk_ref.shape[1])
    x1 = jnp.dot(x0_ref[...], win_ref[...], preferred_element_type=jnp.float32) + bin_ref[...]
    x1_ref[...] = x1

    @pl.when(i == 0)
    def _():
        acc_ref[...] = jnp.zeros_like(acc_ref)

    acc_ref[...] += jax.lax.dot_general(
        x1, B, (((0,), (0,)), ((), ())), preferred_element_type=jnp.float32)

    @pl.when(i == pl.num_programs(0) - 1)
    def _():
        m1_ref[...] = acc_ref[...].T.astype(jnp.bfloat16)


def _pass2_body(pk_ref, x1_ref, m1_ref, w1_ref, b1_ref, w2_ref, b2_ref,
                x2_ref, m2_ref, acc_ref):
    i = pl.program_id(0)
    Bf = _unpack_bits(pk_ref[...])
    y = x1_ref[...] + jnp.dot(Bf, m1_ref[...], preferred_element_type=jnp.float32)
    h = jnp.maximum(jnp.dot(y, w1_ref[...], preferred_element_type=jnp.float32) + b1_ref[...], 0.0)
    x2 = jnp.dot(h, w2_ref[...], preferred_element_type=jnp.float32) + b2_ref[...]
    x2_ref[...] = x2

    @pl.when(i == 0)
    def _():
        acc_ref[...] = jnp.zeros_like(acc_ref)

    acc_ref[...] += jax.lax.dot_general(
        x2.astype(jnp.bfloat16), Bf,
        (((0,), (0,)), ((), ())), preferred_element_type=jnp.float32)

    @pl.when(i == pl.num_programs(0) - 1)
    def _():
        m2_ref[...] = acc_ref[...].T.astype(jnp.bfloat16)


def _pass3_body(n_total, pk_ref, x2_ref, m2_ref, w1_ref, b1_ref, w2_ref, b2_ref,
                wout_ref, bout_ref, out_ref, pool_ref):
    i = pl.program_id(0)
    Bf = _unpack_bits(pk_ref[...])
    y = x2_ref[...] + jnp.dot(Bf, m2_ref[...], preferred_element_type=jnp.float32)
    h = jnp.maximum(jnp.dot(y, w1_ref[...], preferred_element_type=jnp.float32) + b1_ref[...], 0.0)
    x3 = jnp.dot(h, w2_ref[...], preferred_element_type=jnp.float32) + b2_ref[...]

    @pl.when(i == 0)
    def _():
        pool_ref[...] = jnp.zeros_like(pool_ref)

    pool_ref[...] += jnp.sum(x3, axis=0, keepdims=True)

    @pl.when(i == pl.num_programs(0) - 1)
    def _():
        pooled = pool_ref[...] * (1.0 / n_total)
        z = jnp.dot(pooled, wout_ref[...], preferred_element_type=jnp.float32) + bout_ref[...]
        out_ref[...] = jax.nn.sigmoid(z)


def kernel(x_0, incidence_1, W_in, b_in, W1_0, b1_0, W2_0, b2_0,
           W1_1, b1_1, W2_1, b2_1, W_out, b_out):
    N, D = x_0.shape
    E = incidence_1.shape[1]
    EPK = E // 32
    D2 = W1_0.shape[1]
    NB = 1000 if N % 1000 == 0 else 400
    NB2 = 2000 if N % 2000 == 0 else NB

    b_in2 = b_in.reshape(1, -1)
    b1_02 = b1_0.reshape(1, -1)
    b2_02 = b2_0.reshape(1, -1)
    b1_12 = b1_1.reshape(1, -1)
    b2_12 = b2_1.reshape(1, -1)
    b_out2 = b_out.reshape(1, -1)

    params = pltpu.CompilerParams(dimension_semantics=("arbitrary",))

    x1, pk, m1 = pl.pallas_call(
        _pass1_body,
        grid=(N // NB,),
        in_specs=[
            pl.BlockSpec((NB, D), lambda i: (i, 0)),
            pl.BlockSpec((NB, E), lambda i: (i, 0)),
            pl.BlockSpec((D, D), lambda i: (0, 0)),
            pl.BlockSpec((1, D), lambda i: (0, 0)),
        ],
        out_specs=[
            pl.BlockSpec((NB, D), lambda i: (i, 0)),
            pl.BlockSpec((NB, EPK), lambda i: (i, 0)),
            pl.BlockSpec((E, D), lambda i: (0, 0)),
        ],
        out_shape=[
            jax.ShapeDtypeStruct((N, D), jnp.float32),
            jax.ShapeDtypeStruct((N, EPK), jnp.int32),
            jax.ShapeDtypeStruct((E, D), jnp.bfloat16),
        ],
        scratch_shapes=[pltpu.VMEM((D, E), jnp.float32)],
        compiler_params=params,
    )(x_0, incidence_1, W_in, b_in2)

    x2, m2 = pl.pallas_call(
        _pass2_body,
        grid=(N // NB2,),
        in_specs=[
            pl.BlockSpec((NB2, EPK), lambda i: (i, 0)),
            pl.BlockSpec((NB2, D), lambda i: (i, 0)),
            pl.BlockSpec((E, D), lambda i: (0, 0)),
            pl.BlockSpec((D, D2), lambda i: (0, 0)),
            pl.BlockSpec((1, D2), lambda i: (0, 0)),
            pl.BlockSpec((D2, D), lambda i: (0, 0)),
            pl.BlockSpec((1, D), lambda i: (0, 0)),
        ],
        out_specs=[
            pl.BlockSpec((NB2, D), lambda i: (i, 0)),
            pl.BlockSpec((E, D), lambda i: (0, 0)),
        ],
        out_shape=[
            jax.ShapeDtypeStruct((N, D), jnp.float32),
            jax.ShapeDtypeStruct((E, D), jnp.bfloat16),
        ],
        scratch_shapes=[pltpu.VMEM((D, E), jnp.float32)],
        compiler_params=params,
    )(pk, x1, m1, W1_0, b1_02, W2_0, b2_02)

    out = pl.pallas_call(
        functools.partial(_pass3_body, float(N)),
        grid=(N // NB2,),
        in_specs=[
            pl.BlockSpec((NB2, EPK), lambda i: (i, 0)),
            pl.BlockSpec((NB2, D), lambda i: (i, 0)),
            pl.BlockSpec((E, D), lambda i: (0, 0)),
            pl.BlockSpec((D, D2), lambda i: (0, 0)),
            pl.BlockSpec((1, D2), lambda i: (0, 0)),
            pl.BlockSpec((D2, D), lambda i: (0, 0)),
            pl.BlockSpec((1, D), lambda i: (0, 0)),
            pl.BlockSpec((D, W_out.shape[1]), lambda i: (0, 0)),
            pl.BlockSpec((1, W_out.shape[1]), lambda i: (0, 0)),
        ],
        out_specs=pl.BlockSpec((1, W_out.shape[1]), lambda i: (0, 0)),
        out_shape=jax.ShapeDtypeStruct((1, W_out.shape[1]), jnp.float32),
        scratch_shapes=[pltpu.VMEM((1, D), jnp.float32)],
        compiler_params=params,
    )(pk, x2, m2, W1_1, b1_12, W2_1, b2_12, W_out, b_out2)

    return out.reshape(-1)
